# Initial kernel scaffold; baseline (speedup 1.0000x reference)
import jax
import jax.numpy as jnp
from jax.experimental import pallas as pl


def kernel(node_features, node_attr, edge_src, edge_dst, edge_attr, edge_scalars, W_sc_1, W_lin1_1, fc_w1_1, fc_w2_1, W_alpha_1, W_lin2_1, W_sc_2, W_lin1_2, fc_w1_2, fc_w2_2, W_alpha_2, W_lin2_2):
    raise NotImplementedError("write your pallas kernel here")


# speedup vs baseline: 1.8406x; 1.8406x over previous
import functools

import jax
import jax.numpy as jnp
import numpy as np
from jax import lax
from jax.experimental import pallas as pl
from jax.experimental.pallas import tpu as pltpu
from jax.experimental.pallas import tpu_sc as plsc

N = 10000
E = 320000
D = 128
DA = 8
DP = 144
NC = 2
NS = 16
NW = NC * NS
EPW = E // NW
K = 80
NBLK = EPW // K
NP = 10240
RPT = NP // NS

BN = 400
BE = 2560


def _pre_body(x_ref, a_ref, w_ref, sc_ref, nfp_ref):
    x = x_ref[...]
    a = a_ref[...]
    z = jnp.concatenate([a[:, j:j + 1] * x for j in range(DA)], axis=1)
    r = jnp.dot(z, w_ref[...], preferred_element_type=jnp.float32)
    sc_ref[...] = r[:, :D]
    pad = jnp.concatenate(
        [jnp.ones((x.shape[0], 1), jnp.float32),
         jnp.zeros((x.shape[0], DP - D - 1), jnp.float32)], axis=1)
    nfp_ref[...] = jnp.concatenate([r[:, D:], pad], axis=1)


def _pre_call(x, attr, wflat):
    return pl.pallas_call(
        _pre_body,
        grid=(N // BN,),
        in_specs=[
            pl.BlockSpec((BN, D), lambda i: (i, 0)),
            pl.BlockSpec((BN, DA), lambda i: (i, 0)),
            pl.BlockSpec((D * DA, 2 * D), lambda i: (0, 0)),
        ],
        out_specs=[
            pl.BlockSpec((BN, D), lambda i: (i, 0)),
            pl.BlockSpec((BN, DP), lambda i: (i, 0)),
        ],
        out_shape=[
            jax.ShapeDtypeStruct((N, D), jnp.float32),
            jax.ShapeDtypeStruct((N, DP), jnp.float32),
        ],
    )(x, attr, wflat)


def _edge_mlp_body(es_ref, ea_ref, w1_ref, w2_ref, out_ref):
    h = jnp.dot(es_ref[...], w1_ref[...], preferred_element_type=jnp.float32)
    h = h * jax.nn.sigmoid(h)
    w = jnp.dot(h, w2_ref[...], preferred_element_type=jnp.float32)
    out_ref[...] = w * ea_ref[...]


def _edge_mlp_call(es, ea, w1, w2):
    nb = es.shape[1]
    return pl.pallas_call(
        _edge_mlp_body,
        grid=(E // BE,),
        in_specs=[
            pl.BlockSpec((BE, nb), lambda i: (i, 0)),
            pl.BlockSpec((BE, 1), lambda i: (i, 0)),
            pl.BlockSpec((nb, 64), lambda i: (0, 0)),
            pl.BlockSpec((64, D), lambda i: (0, 0)),
        ],
        out_specs=pl.BlockSpec((BE, D), lambda i: (i, 0)),
        out_shape=jax.ShapeDtypeStruct((E, D), jnp.float32),
    )(es, ea, w1, w2)


def _edge_sc_body(nfp_hbm, src_hbm, dst_hbm, ew_hbm, z_hbm, out_hbm,
                  sidx, didx, rows, ewb, acc, sem):
    c = lax.axis_index("c")
    s = lax.axis_index("s")
    wid = s * NC + c

    pltpu.sync_copy(z_hbm, acc.at[pl.ds(s * RPT, RPT)])
    plsc.subcore_barrier()

    ebase = wid * EPW

    def body(b, carry):
        base = ebase + b * K
        pltpu.sync_copy(src_hbm.at[pl.ds(base, K)], sidx)
        pltpu.sync_copy(dst_hbm.at[pl.ds(base, K)], didx)
        gat = pltpu.async_copy(nfp_hbm.at[sidx], rows, sem)
        pltpu.sync_copy(ew_hbm.at[pl.ds(base, K)], ewb)
        gat.wait()

        def mul(k, carry2):
            for j in range(D // 16):
                sl = pl.ds(j * 16, 16)
                rows[k, sl] = rows[k, sl] * ewb[k, sl]
            return carry2

        lax.fori_loop(0, K, mul, 0, unroll=2)
        pltpu.sync_copy(rows, acc.at[didx], add=True)
        return carry

    lax.fori_loop(0, NBLK, body, 0)
    plsc.subcore_barrier()

    rbase = s * RPT
    pltpu.sync_copy(acc.at[pl.ds(rbase, RPT)], out_hbm.at[c, pl.ds(rbase, RPT)])


@functools.cache
def _edge_sc():
    mesh = plsc.VectorSubcoreMesh(
        core_axis_name="c", subcore_axis_name="s",
        num_cores=NC, num_subcores=NS)
    return pl.kernel(
        _edge_sc_body,
        out_type=jax.ShapeDtypeStruct((NC, NP, DP), jnp.float32),
        mesh=mesh,
        scratch_types=[
            pltpu.VMEM((K,), jnp.int32),
            pltpu.VMEM((K,), jnp.int32),
            pltpu.VMEM((K, DP), jnp.float32),
            pltpu.VMEM((K, D), jnp.float32),
            pltpu.VMEM_SHARED((NP, DP), jnp.float32),
            pltpu.SemaphoreType.DMA,
        ],
        compiler_params=pltpu.CompilerParams(use_tc_tiling_on_sc=False),
    )


def _post_body(ap_ref, sc_ref, a_ref, w_ref, wa_ref, out_ref, *, act):
    aps = ap_ref[0] + ap_ref[1]
    cnt = jnp.maximum(aps[:, D:D + 1], 1.0)
    agg = aps[:, :D] / cnt
    a = a_ref[...]
    z = jnp.concatenate([a[:, j:j + 1] * agg for j in range(DA)], axis=1)
    om = jnp.dot(z, w_ref[...], preferred_element_type=jnp.float32)
    am = jnp.dot(a, wa_ref[...], preferred_element_type=jnp.float32)
    alpha = jnp.sum(agg * am, axis=1, keepdims=True)
    y = sc_ref[...] + alpha * om
    if act:
        y = y * jax.nn.sigmoid(y)
    out_ref[...] = y


def _post_call(aggp, sc, attr, wflat, wa, act):
    return pl.pallas_call(
        functools.partial(_post_body, act=act),
        grid=(N // BN,),
        in_specs=[
            pl.BlockSpec((NC, BN, DP), lambda i: (0, i, 0)),
            pl.BlockSpec((BN, D), lambda i: (i, 0)),
            pl.BlockSpec((BN, DA), lambda i: (i, 0)),
            pl.BlockSpec((D * DA, D), lambda i: (0, 0)),
            pl.BlockSpec((DA, D), lambda i: (0, 0)),
        ],
        out_specs=pl.BlockSpec((BN, D), lambda i: (i, 0)),
        out_shape=jax.ShapeDtypeStruct((N, D), jnp.float32),
    )(aggp, sc, attr, wflat, wa)


def kernel(node_features, node_attr, edge_src, edge_dst, edge_attr,
           edge_scalars,
           W_sc_1, W_lin1_1, fc_w1_1, fc_w2_1, W_alpha_1, W_lin2_1,
           W_sc_2, W_lin1_2, fc_w1_2, fc_w2_2, W_alpha_2, W_lin2_2):
    src = edge_src.astype(jnp.int32)
    dst = edge_dst.astype(jnp.int32)
    attr = node_attr
    zrows = jnp.zeros((RPT, DP), jnp.float32)
    s_tp = 1.0 / np.sqrt(D * DA)

    x = node_features
    layers = (
        (W_sc_1, W_lin1_1, fc_w1_1, fc_w2_1, W_alpha_1, W_lin2_1, True),
        (W_sc_2, W_lin1_2, fc_w1_2, fc_w2_2, W_alpha_2, W_lin2_2, False),
    )
    for Wsc, Wl1, w1, w2, Wa, Wl2, act in layers:
        wcat = (jnp.concatenate([Wsc, Wl1], axis=2)
                .transpose(1, 0, 2).reshape(D * DA, 2 * D) * s_tp)
        sc, nfp = _pre_call(x, attr, wcat)
        ew = _edge_mlp_call(edge_scalars, edge_attr,
                            w1 / np.sqrt(w1.shape[0]),
                            w2 / np.sqrt(w2.shape[0]))
        aggp = _edge_sc()(nfp, src, dst, ew, zrows)
        w2f = Wl2.transpose(1, 0, 2).reshape(D * DA, D) * s_tp
        wa = Wa[:, :, 0].T * s_tp
        x = _post_call(aggp, sc, attr, w2f, wa, act)
    return x


# speedup vs baseline: 2.1938x; 1.1919x over previous
import functools

import jax
import jax.numpy as jnp
import numpy as np
from jax import lax
from jax.experimental import pallas as pl
from jax.experimental.pallas import tpu as pltpu
from jax.experimental.pallas import tpu_sc as plsc

N = 10000
E = 320000
D = 128
DA = 8
DP = 144
NC = 2
NS = 16
NW = NC * NS
EPW = E // NW
K = 80
NBLK = EPW // K
NP = 10240
RPT = NP // NS

BN = 400
BE = 2560


def _pre_body(x_ref, a_ref, w_ref, sc_ref, nfp_ref):
    x = x_ref[...]
    a = a_ref[...]
    z = jnp.concatenate([a[:, j:j + 1] * x for j in range(DA)], axis=1)
    r = jnp.dot(z, w_ref[...], preferred_element_type=jnp.float32)
    sc_ref[...] = r[:, :D]
    pad = jnp.concatenate(
        [jnp.ones((x.shape[0], 1), jnp.float32),
         jnp.zeros((x.shape[0], DP - D - 1), jnp.float32)], axis=1)
    nfp_ref[...] = jnp.concatenate([r[:, D:], pad], axis=1)


def _pre_call(x, attr, wflat):
    return pl.pallas_call(
        _pre_body,
        grid=(N // BN,),
        in_specs=[
            pl.BlockSpec((BN, D), lambda i: (i, 0)),
            pl.BlockSpec((BN, DA), lambda i: (i, 0)),
            pl.BlockSpec((D * DA, 2 * D), lambda i: (0, 0)),
        ],
        out_specs=[
            pl.BlockSpec((BN, D), lambda i: (i, 0)),
            pl.BlockSpec((BN, DP), lambda i: (i, 0)),
        ],
        out_shape=[
            jax.ShapeDtypeStruct((N, D), jnp.float32),
            jax.ShapeDtypeStruct((N, DP), jnp.float32),
        ],
    )(x, attr, wflat)


def _edge_mlp_body(es_ref, ea_ref, w1_ref, w2_ref, out_ref):
    h = jnp.dot(es_ref[...], w1_ref[...], preferred_element_type=jnp.float32)
    h = h * jax.nn.sigmoid(h)
    w = jnp.dot(h, w2_ref[...], preferred_element_type=jnp.float32)
    out_ref[...] = (w * ea_ref[...]).astype(jnp.bfloat16)


def _edge_mlp_call(es, ea, w1, w2):
    nb = es.shape[1]
    return pl.pallas_call(
        _edge_mlp_body,
        grid=(E // BE,),
        in_specs=[
            pl.BlockSpec((BE, nb), lambda i: (i, 0)),
            pl.BlockSpec((BE, 1), lambda i: (i, 0)),
            pl.BlockSpec((nb, 64), lambda i: (0, 0)),
            pl.BlockSpec((64, D), lambda i: (0, 0)),
        ],
        out_specs=pl.BlockSpec((BE, D), lambda i: (i, 0)),
        out_shape=jax.ShapeDtypeStruct((E, D), jnp.bfloat16),
    )(es, ea, w1, w2)


_EW_PERM = np.array([32 * (c // 32) + 16 * (c % 2) + (c % 32) // 2
                     for c in range(D)])


def _edge_sc_body(nfp_hbm, sd_hbm, ew_hbm, z_hbm, out_hbm,
                  ii0, ii1, ii2, ii3, rows0, rows1, ewb0, ewb1, acc,
                  is0, is1, is2, is3, gs0, gs1, es0, es1, ss0, ss1):
    iidx = (ii0, ii1, ii2, ii3)
    isem = (is0, is1, is2, is3)
    rows = (rows0, rows1)
    ewb = (ewb0, ewb1)
    gs = (gs0, gs1)
    es = (es0, es1)
    ss = (ss0, ss1)
    c = lax.axis_index("c")
    s = lax.axis_index("s")
    wid = s * NC + c

    pltpu.sync_copy(z_hbm, acc.at[pl.ds(s * RPT, RPT)])
    plsc.subcore_barrier()

    ebase = wid * EPW

    def load_idx(b, r):
        pltpu.async_copy(sd_hbm.at[wid, b], iidx[r], isem[r])

    def wait_idx(r):
        pltpu.make_async_copy(sd_hbm.at[wid, 0], iidx[r], isem[r]).wait()

    def issue_in(b, p, r):
        pltpu.async_copy(nfp_hbm.at[iidx[r].at[0]], rows[p], gs[p])
        pltpu.async_copy(ew_hbm.at[pl.ds(ebase + b * K, K)], ewb[p], es[p])

    def wait_in(p):
        pltpu.make_async_copy(nfp_hbm.at[ii0.at[0]], rows[p], gs[p]).wait()
        pltpu.make_async_copy(ew_hbm.at[pl.ds(0, K)], ewb[p], es[p]).wait()

    def wait_sc(p):
        pltpu.make_async_copy(rows[p], acc.at[ii0.at[1]], ss[p]).wait()

    def mul(p):
        def body(k, carry):
            for j2 in range(D // 32):
                eb = ewb[p][k, pl.ds(32 * j2, 32)]
                e0, e1 = plsc.unpack(eb, format=plsc.PackFormat.INTERLEAVED)
                s0 = pl.ds(32 * j2, 16)
                s1 = pl.ds(32 * j2 + 16, 16)
                rows[p][k, s0] = rows[p][k, s0] * e0
                rows[p][k, s1] = rows[p][k, s1] * e1
            return carry
        lax.fori_loop(0, K, body, 0, unroll=2)

    pltpu.sync_copy(sd_hbm.at[wid, 0], ii0)
    issue_in(0, 0, 0)
    load_idx(1, 1)

    def body_step(b, j, first):
        p = j % 2
        q = 1 - p
        wait_in(p)
        mul(p)
        pltpu.async_copy(rows[p], acc.at[iidx[j].at[1]], ss[p], add=True)
        if first is None:
            wait_sc(q)
        else:
            @pl.when(first)
            def _():
                wait_sc(q)
        wait_idx((j + 1) % 4)
        issue_in(b + 1, q, (j + 1) % 4)
        load_idx(b + 2, (j + 2) % 4)

    def outer(g, carry):
        for j in range(4):
            b = 4 * g + j
            if j == 0:
                body_step(b, j, g >= 1)
            else:
                body_step(b, j, None)
        return carry

    lax.fori_loop(0, (NBLK - 5) // 4, outer, 0)

    for b in range(NBLK - 5, NBLK):
        j = b % 4
        p = j % 2
        wait_in(p)
        mul(p)
        pltpu.async_copy(rows[p], acc.at[iidx[j].at[1]], ss[p], add=True)
        wait_sc(1 - p)
        if b + 1 < NBLK:
            wait_idx((j + 1) % 4)
            issue_in(b + 1, 1 - p, (j + 1) % 4)
        if b + 2 < NBLK:
            load_idx(b + 2, (j + 2) % 4)
    wait_sc((NBLK - 1) % 2)
    plsc.subcore_barrier()

    rbase = s * RPT
    pltpu.sync_copy(acc.at[pl.ds(rbase, RPT)], out_hbm.at[c, pl.ds(rbase, RPT)])


@functools.cache
def _edge_sc():
    mesh = plsc.VectorSubcoreMesh(
        core_axis_name="c", subcore_axis_name="s",
        num_cores=NC, num_subcores=NS)
    return pl.kernel(
        _edge_sc_body,
        out_type=jax.ShapeDtypeStruct((NC, NP, DP), jnp.float32),
        mesh=mesh,
        scratch_types=(
            [pltpu.VMEM((2, K), jnp.int32)] * 4
            + [pltpu.VMEM((K, DP), jnp.float32)] * 2
            + [pltpu.VMEM((K, D), jnp.bfloat16)] * 2
            + [pltpu.VMEM_SHARED((NP, DP), jnp.float32)]
            + [pltpu.SemaphoreType.DMA] * 10
        ),
        compiler_params=pltpu.CompilerParams(
            use_tc_tiling_on_sc=False, needs_layout_passes=False),
    )


def _post_body(ap_ref, sc_ref, a_ref, w_ref, wa_ref, out_ref, *, act):
    aps = ap_ref[0] + ap_ref[1]
    cnt = jnp.maximum(aps[:, D:D + 1], 1.0)
    agg = aps[:, :D] / cnt
    a = a_ref[...]
    z = jnp.concatenate([a[:, j:j + 1] * agg for j in range(DA)], axis=1)
    om = jnp.dot(z, w_ref[...], preferred_element_type=jnp.float32)
    am = jnp.dot(a, wa_ref[...], preferred_element_type=jnp.float32)
    alpha = jnp.sum(agg * am, axis=1, keepdims=True)
    y = sc_ref[...] + alpha * om
    if act:
        y = y * jax.nn.sigmoid(y)
    out_ref[...] = y


def _post_call(aggp, sc, attr, wflat, wa, act):
    return pl.pallas_call(
        functools.partial(_post_body, act=act),
        grid=(N // BN,),
        in_specs=[
            pl.BlockSpec((NC, BN, DP), lambda i: (0, i, 0)),
            pl.BlockSpec((BN, D), lambda i: (i, 0)),
            pl.BlockSpec((BN, DA), lambda i: (i, 0)),
            pl.BlockSpec((D * DA, D), lambda i: (0, 0)),
            pl.BlockSpec((DA, D), lambda i: (0, 0)),
        ],
        out_specs=pl.BlockSpec((BN, D), lambda i: (i, 0)),
        out_shape=jax.ShapeDtypeStruct((N, D), jnp.float32),
    )(aggp, sc, attr, wflat, wa)


def kernel(node_features, node_attr, edge_src, edge_dst, edge_attr,
           edge_scalars,
           W_sc_1, W_lin1_1, fc_w1_1, fc_w2_1, W_alpha_1, W_lin2_1,
           W_sc_2, W_lin1_2, fc_w1_2, fc_w2_2, W_alpha_2, W_lin2_2):
    sd = jnp.stack([edge_src.astype(jnp.int32).reshape(NW, NBLK, K),
                    edge_dst.astype(jnp.int32).reshape(NW, NBLK, K)], axis=2)
    attr = node_attr
    zrows = jnp.zeros((RPT, DP), jnp.float32)
    s_tp = 1.0 / np.sqrt(D * DA)

    x = node_features
    layers = (
        (W_sc_1, W_lin1_1, fc_w1_1, fc_w2_1, W_alpha_1, W_lin2_1, True),
        (W_sc_2, W_lin1_2, fc_w1_2, fc_w2_2, W_alpha_2, W_lin2_2, False),
    )
    for Wsc, Wl1, w1, w2, Wa, Wl2, act in layers:
        wcat = (jnp.concatenate([Wsc, Wl1], axis=2)
                .transpose(1, 0, 2).reshape(D * DA, 2 * D) * s_tp)
        sc, nfp = _pre_call(x, attr, wcat)
        ew = _edge_mlp_call(edge_scalars, edge_attr,
                            w1 / np.sqrt(w1.shape[0]),
                            w2[:, _EW_PERM] / np.sqrt(w2.shape[0]))
        aggp = _edge_sc()(nfp, sd, ew, zrows)
        w2f = Wl2.transpose(1, 0, 2).reshape(D * DA, D) * s_tp
        wa = Wa[:, :, 0].T * s_tp
        x = _post_call(aggp, sc, attr, w2f, wa, act)
    return x


# speedup vs baseline: 2.6304x; 1.1990x over previous
import functools

import jax
import jax.numpy as jnp
import numpy as np
from jax import lax
from jax.experimental import pallas as pl
from jax.experimental.pallas import tpu as pltpu
from jax.experimental.pallas import tpu_sc as plsc

N = 10000
E = 320000
D = 128
DA = 8
DP = 144
NC = 2
NS = 16
NW = NC * NS
EPW = E // NW
K = 80
NBLK = EPW // K
NP = 10240
RPT = NP // NS

BN = 400
BE = 2560


def _pre_body(x_ref, a_ref, w_ref, sc_ref, nfp_ref):
    x = x_ref[...]
    a = a_ref[...]
    z = jnp.concatenate([a[:, j:j + 1] * x for j in range(DA)], axis=1)
    r = jnp.dot(z, w_ref[...], preferred_element_type=jnp.float32)
    sc_ref[...] = r[:, :D]
    pad = jnp.concatenate(
        [jnp.ones((x.shape[0], 1), jnp.float32),
         jnp.zeros((x.shape[0], DP - D - 1), jnp.float32)], axis=1)
    nfp_ref[...] = jnp.concatenate([r[:, D:], pad], axis=1)


def _pre_call(x, attr, wflat):
    return pl.pallas_call(
        _pre_body,
        grid=(N // BN,),
        in_specs=[
            pl.BlockSpec((BN, D), lambda i: (i, 0)),
            pl.BlockSpec((BN, DA), lambda i: (i, 0)),
            pl.BlockSpec((D * DA, 2 * D), lambda i: (0, 0)),
        ],
        out_specs=[
            pl.BlockSpec((BN, D), lambda i: (i, 0)),
            pl.BlockSpec((BN, DP), lambda i: (i, 0)),
        ],
        out_shape=[
            jax.ShapeDtypeStruct((N, D), jnp.float32),
            jax.ShapeDtypeStruct((N, DP), jnp.float32),
        ],
    )(x, attr, wflat)


def _edge_mlp_body(es_ref, ea_ref, w1_ref, w2_ref, out_ref):
    h = jnp.dot(es_ref[...], w1_ref[...], preferred_element_type=jnp.float32)
    h = h * jax.nn.sigmoid(h)
    w = jnp.dot(h, w2_ref[...], preferred_element_type=jnp.float32)
    wb = (w * ea_ref[...]).astype(jnp.bfloat16)
    wr = wb.reshape(BE // 16, 2, 8, D)
    lo = jax.lax.bitcast_convert_type(wr[:, 0], jnp.uint16).astype(jnp.uint32)
    hi = jax.lax.bitcast_convert_type(wr[:, 1], jnp.uint16).astype(jnp.uint32)
    out_ref[...] = (lo | (hi << 16)).reshape(BE // 2, D)


def _edge_mlp_call(es, ea, w1, w2):
    nb = es.shape[1]
    return pl.pallas_call(
        _edge_mlp_body,
        grid=(E // BE,),
        in_specs=[
            pl.BlockSpec((BE, nb), lambda i: (i, 0)),
            pl.BlockSpec((BE, 1), lambda i: (i, 0)),
            pl.BlockSpec((nb, 64), lambda i: (0, 0)),
            pl.BlockSpec((64, D), lambda i: (0, 0)),
        ],
        out_specs=pl.BlockSpec((BE // 2, D), lambda i: (i, 0)),
        out_shape=jax.ShapeDtypeStruct((E // 2, D), jnp.uint32),
    )(es, ea, w1, w2)


def _edge_sc_body(nfp_hbm, sd_hbm, ew_hbm, z_hbm, out_hbm,
                  ii0, ii1, ii2, ii3, rows0, rows1, ewb0, ewb1, acc,
                  is0, is1, is2, is3, gs0, gs1, es0, es1, ss0, ss1):
    iidx = (ii0, ii1, ii2, ii3)
    isem = (is0, is1, is2, is3)
    rows = (rows0, rows1)
    ewb = (ewb0, ewb1)
    gs = (gs0, gs1)
    es = (es0, es1)
    ss = (ss0, ss1)
    c = lax.axis_index("c")
    s = lax.axis_index("s")
    wid = s * NC + c

    pltpu.sync_copy(z_hbm, acc.at[pl.ds(s * RPT, RPT)])
    plsc.subcore_barrier()

    ebase = wid * EPW

    def load_idx(b, r):
        pltpu.async_copy(sd_hbm.at[wid, b], iidx[r], isem[r])

    def wait_idx(r):
        pltpu.make_async_copy(sd_hbm.at[wid, 0], iidx[r], isem[r]).wait()

    def issue_in(b, p, r):
        pltpu.async_copy(nfp_hbm.at[iidx[r].at[0]], rows[p], gs[p])
        pltpu.async_copy(ew_hbm.at[pl.ds((ebase + b * K) // 2, K // 2)],
                         ewb[p], es[p])

    def wait_in(p):
        pltpu.make_async_copy(nfp_hbm.at[ii0.at[0]], rows[p], gs[p]).wait()
        pltpu.make_async_copy(ew_hbm.at[pl.ds(0, K // 2)], ewb[p], es[p]).wait()

    def wait_sc(p):
        pltpu.make_async_copy(rows[p], acc.at[ii0.at[1]], ss[p]).wait()

    def mul(p):
        def body(m, carry):
            for s in range(8):
                for j in range(D // 16):
                    ww = ewb[p][8 * m + s, pl.ds(16 * j, 16)]
                    eb = plsc.bitcast(ww, jnp.bfloat16)
                    e0, e1 = plsc.unpack(eb, format=plsc.PackFormat.INTERLEAVED)
                    sl = pl.ds(16 * j, 16)
                    ra = 16 * m + s
                    rb = 16 * m + 8 + s
                    rows[p][ra, sl] = rows[p][ra, sl] * e0
                    rows[p][rb, sl] = rows[p][rb, sl] * e1
            return carry
        lax.fori_loop(0, K // 16, body, 0)

    pltpu.sync_copy(sd_hbm.at[wid, 0], ii0)
    issue_in(0, 0, 0)
    load_idx(1, 1)

    def body_step(b, j, first):
        p = j % 2
        q = 1 - p
        wait_in(p)
        mul(p)
        pltpu.async_copy(rows[p], acc.at[iidx[j].at[1]], ss[p], add=True)
        if first is None:
            wait_sc(q)
        else:
            @pl.when(first)
            def _():
                wait_sc(q)
        wait_idx((j + 1) % 4)
        issue_in(b + 1, q, (j + 1) % 4)
        load_idx(b + 2, (j + 2) % 4)

    def outer(g, carry):
        for j in range(4):
            b = 4 * g + j
            if j == 0:
                body_step(b, j, g >= 1)
            else:
                body_step(b, j, None)
        return carry

    lax.fori_loop(0, (NBLK - 5) // 4, outer, 0)

    for b in range(NBLK - 5, NBLK):
        j = b % 4
        p = j % 2
        wait_in(p)
        mul(p)
        pltpu.async_copy(rows[p], acc.at[iidx[j].at[1]], ss[p], add=True)
        wait_sc(1 - p)
        if b + 1 < NBLK:
            wait_idx((j + 1) % 4)
            issue_in(b + 1, 1 - p, (j + 1) % 4)
        if b + 2 < NBLK:
            load_idx(b + 2, (j + 2) % 4)
    wait_sc((NBLK - 1) % 2)
    plsc.subcore_barrier()

    rbase = s * RPT
    pltpu.sync_copy(acc.at[pl.ds(rbase, RPT)], out_hbm.at[c, pl.ds(rbase, RPT)])


@functools.cache
def _edge_sc():
    mesh = plsc.VectorSubcoreMesh(
        core_axis_name="c", subcore_axis_name="s",
        num_cores=NC, num_subcores=NS)
    return pl.kernel(
        _edge_sc_body,
        out_type=jax.ShapeDtypeStruct((NC, NP, DP), jnp.float32),
        mesh=mesh,
        scratch_types=(
            [pltpu.VMEM((2, K), jnp.int32)] * 4
            + [pltpu.VMEM((K, DP), jnp.float32)] * 2
            + [pltpu.VMEM((K // 2, D), jnp.uint32)] * 2
            + [pltpu.VMEM_SHARED((NP, DP), jnp.float32)]
            + [pltpu.SemaphoreType.DMA] * 10
        ),
        compiler_params=pltpu.CompilerParams(
            use_tc_tiling_on_sc=False, needs_layout_passes=False),
    )


def _post_body(ap_ref, sc_ref, a_ref, w_ref, wa_ref, out_ref, *, act):
    aps = ap_ref[0] + ap_ref[1]
    cnt = jnp.maximum(aps[:, D:D + 1], 1.0)
    agg = aps[:, :D] / cnt
    a = a_ref[...]
    z = jnp.concatenate([a[:, j:j + 1] * agg for j in range(DA)], axis=1)
    om = jnp.dot(z, w_ref[...], preferred_element_type=jnp.float32)
    am = jnp.dot(a, wa_ref[...], preferred_element_type=jnp.float32)
    alpha = jnp.sum(agg * am, axis=1, keepdims=True)
    y = sc_ref[...] + alpha * om
    if act:
        y = y * jax.nn.sigmoid(y)
    out_ref[...] = y


def _post_call(aggp, sc, attr, wflat, wa, act):
    return pl.pallas_call(
        functools.partial(_post_body, act=act),
        grid=(N // BN,),
        in_specs=[
            pl.BlockSpec((NC, BN, DP), lambda i: (0, i, 0)),
            pl.BlockSpec((BN, D), lambda i: (i, 0)),
            pl.BlockSpec((BN, DA), lambda i: (i, 0)),
            pl.BlockSpec((D * DA, D), lambda i: (0, 0)),
            pl.BlockSpec((DA, D), lambda i: (0, 0)),
        ],
        out_specs=pl.BlockSpec((BN, D), lambda i: (i, 0)),
        out_shape=jax.ShapeDtypeStruct((N, D), jnp.float32),
    )(aggp, sc, attr, wflat, wa)


def kernel(node_features, node_attr, edge_src, edge_dst, edge_attr,
           edge_scalars,
           W_sc_1, W_lin1_1, fc_w1_1, fc_w2_1, W_alpha_1, W_lin2_1,
           W_sc_2, W_lin1_2, fc_w1_2, fc_w2_2, W_alpha_2, W_lin2_2):
    sd = jnp.stack([edge_src.astype(jnp.int32).reshape(NW, NBLK, K),
                    edge_dst.astype(jnp.int32).reshape(NW, NBLK, K)], axis=2)
    attr = node_attr
    zrows = jnp.zeros((RPT, DP), jnp.float32)
    s_tp = 1.0 / np.sqrt(D * DA)

    x = node_features
    layers = (
        (W_sc_1, W_lin1_1, fc_w1_1, fc_w2_1, W_alpha_1, W_lin2_1, True),
        (W_sc_2, W_lin1_2, fc_w1_2, fc_w2_2, W_alpha_2, W_lin2_2, False),
    )
    for Wsc, Wl1, w1, w2, Wa, Wl2, act in layers:
        wcat = (jnp.concatenate([Wsc, Wl1], axis=2)
                .transpose(1, 0, 2).reshape(D * DA, 2 * D) * s_tp)
        sc, nfp = _pre_call(x, attr, wcat)
        ew = _edge_mlp_call(edge_scalars, edge_attr,
                            w1 / np.sqrt(w1.shape[0]),
                            w2 / np.sqrt(w2.shape[0]))
        aggp = _edge_sc()(nfp, sd, ew, zrows)
        w2f = Wl2.transpose(1, 0, 2).reshape(D * DA, D) * s_tp
        wa = Wa[:, :, 0].T * s_tp
        x = _post_call(aggp, sc, attr, w2f, wa, act)
    return x


# speedup vs baseline: 2.7083x; 1.0296x over previous
import functools

import jax
import jax.numpy as jnp
import numpy as np
from jax import lax
from jax.experimental import pallas as pl
from jax.experimental.pallas import tpu as pltpu
from jax.experimental.pallas import tpu_sc as plsc

N = 10000
E = 320000
D = 128
DA = 8
DP = 144
NC = 2
NS = 16
NW = NC * NS
EPW = E // NW
K = 80
NBLK = EPW // K
NP = 10240
RPT = NP // NS

BN = 400
BE = 2560


def _pack2(lo, hi):
    l16 = jax.lax.bitcast_convert_type(lo.astype(jnp.bfloat16), jnp.uint16)
    h16 = jax.lax.bitcast_convert_type(hi.astype(jnp.bfloat16), jnp.uint16)
    return l16.astype(jnp.uint32) | (h16.astype(jnp.uint32) << 16)


def _pre_body(x_ref, a_ref, w_ref, sc_ref, nfp_ref):
    x = x_ref[...]
    a = a_ref[...]
    z = jnp.concatenate([a[:, j:j + 1] * x for j in range(DA)], axis=1)
    r = jnp.dot(z, w_ref[...], preferred_element_type=jnp.float32)
    sc_ref[...] = r[:, :D]
    nf = r[:, D:]
    nfp_ref[...] = _pack2(nf[:, :D // 2], nf[:, D // 2:])


def _pre_call(x, attr, wflat):
    return pl.pallas_call(
        _pre_body,
        grid=(N // BN,),
        in_specs=[
            pl.BlockSpec((BN, D), lambda i: (i, 0)),
            pl.BlockSpec((BN, DA), lambda i: (i, 0)),
            pl.BlockSpec((D * DA, 2 * D), lambda i: (0, 0)),
        ],
        out_specs=[
            pl.BlockSpec((BN, D), lambda i: (i, 0)),
            pl.BlockSpec((BN, D // 2), lambda i: (i, 0)),
        ],
        out_shape=[
            jax.ShapeDtypeStruct((N, D), jnp.float32),
            jax.ShapeDtypeStruct((N, D // 2), jnp.uint32),
        ],
    )(x, attr, wflat)


def _edge_mlp_body(es_ref, ea_ref, w1_ref, w2_ref, out_ref):
    h = jnp.dot(es_ref[...], w1_ref[...], preferred_element_type=jnp.float32)
    h = h * jax.nn.sigmoid(h)
    w = jnp.dot(h, w2_ref[...], preferred_element_type=jnp.float32)
    w = w * ea_ref[...]
    out_ref[...] = _pack2(w[:, :D // 2], w[:, D // 2:])


def _edge_mlp_call(es, ea, w1, w2):
    nb = es.shape[1]
    return pl.pallas_call(
        _edge_mlp_body,
        grid=(E // BE,),
        in_specs=[
            pl.BlockSpec((BE, nb), lambda i: (i, 0)),
            pl.BlockSpec((BE, 1), lambda i: (i, 0)),
            pl.BlockSpec((nb, 64), lambda i: (0, 0)),
            pl.BlockSpec((64, D), lambda i: (0, 0)),
        ],
        out_specs=pl.BlockSpec((BE, D // 2), lambda i: (i, 0)),
        out_shape=jax.ShapeDtypeStruct((E, D // 2), jnp.uint32),
    )(es, ea, w1, w2)


def _edge_sc_body(nfp_hbm, sd_hbm, ew_hbm, z_hbm, out_hbm,
                  ii0, ii1, ii2, ii3, rows0, rows1, ewb0, ewb1, srows, acc,
                  is0, is1, is2, is3, gs0, gs1, es0, es1, ss0):
    iidx = (ii0, ii1, ii2, ii3)
    isem = (is0, is1, is2, is3)
    rows = (rows0, rows1)
    ewb = (ewb0, ewb1)
    gs = (gs0, gs1)
    es = (es0, es1)
    c = lax.axis_index("c")
    s = lax.axis_index("s")
    wid = s * NC + c

    pltpu.sync_copy(z_hbm, acc.at[pl.ds(s * RPT, RPT)])
    cpad = jnp.where(lax.iota(jnp.int32, 16) == 0, 1.0, 0.0)

    def preset(k, carry):
        srows[k, pl.ds(D, 16)] = cpad
        return carry

    lax.fori_loop(0, K, preset, 0)
    plsc.subcore_barrier()

    ebase = wid * EPW

    def load_idx(b, r):
        pltpu.async_copy(sd_hbm.at[wid, b], iidx[r], isem[r])

    def wait_idx(r):
        pltpu.make_async_copy(sd_hbm.at[wid, 0], iidx[r], isem[r]).wait()

    def issue_in(b, p, r):
        pltpu.async_copy(nfp_hbm.at[iidx[r].at[0]], rows[p], gs[p])
        pltpu.async_copy(ew_hbm.at[pl.ds(ebase + b * K, K)], ewb[p], es[p])

    def wait_in(p):
        pltpu.make_async_copy(nfp_hbm.at[ii0.at[0]], rows[p], gs[p]).wait()
        pltpu.make_async_copy(ew_hbm.at[pl.ds(0, K)], ewb[p], es[p]).wait()

    def wait_sc():
        pltpu.make_async_copy(srows, acc.at[ii0.at[1]], ss0).wait()

    def mul(p):
        def body(k, carry):
            for j2 in range(D // 32):
                rw = plsc.bitcast(rows[p][k, pl.ds(16 * j2, 16)], jnp.bfloat16)
                r0, r1 = plsc.unpack(rw, format=plsc.PackFormat.INTERLEAVED)
                ww = plsc.bitcast(ewb[p][k, pl.ds(16 * j2, 16)], jnp.bfloat16)
                e0, e1 = plsc.unpack(ww, format=plsc.PackFormat.INTERLEAVED)
                srows[k, pl.ds(16 * j2, 16)] = r0 * e0
                srows[k, pl.ds(D // 2 + 16 * j2, 16)] = r1 * e1
            return carry
        lax.fori_loop(0, K, body, 0, unroll=2)

    pltpu.sync_copy(sd_hbm.at[wid, 0], ii0)
    issue_in(0, 0, 0)
    load_idx(1, 1)

    def body_step(b, j, notfirst):
        p = j % 2
        q = 1 - p
        wait_in(p)
        wait_idx((j + 1) % 4)
        issue_in(b + 1, q, (j + 1) % 4)
        if notfirst is None:
            wait_sc()
        else:
            @pl.when(notfirst)
            def _():
                wait_sc()
        load_idx(b + 2, (j + 2) % 4)
        mul(p)
        pltpu.async_copy(srows, acc.at[iidx[j].at[1]], ss0, add=True)

    def outer(g, carry):
        for j in range(4):
            b = 4 * g + j
            if j == 0:
                body_step(b, j, g >= 1)
            else:
                body_step(b, j, None)
        return carry

    lax.fori_loop(0, (NBLK - 5) // 4, outer, 0)

    for b in range(NBLK - 5, NBLK):
        j = b % 4
        p = j % 2
        wait_in(p)
        if b + 1 < NBLK:
            wait_idx((j + 1) % 4)
            issue_in(b + 1, 1 - p, (j + 1) % 4)
        wait_sc()
        if b + 2 < NBLK:
            load_idx(b + 2, (j + 2) % 4)
        mul(p)
        pltpu.async_copy(srows, acc.at[iidx[j].at[1]], ss0, add=True)
    wait_sc()
    plsc.subcore_barrier()

    rbase = s * RPT
    pltpu.sync_copy(acc.at[pl.ds(rbase, RPT)], out_hbm.at[c, pl.ds(rbase, RPT)])


@functools.cache
def _edge_sc():
    mesh = plsc.VectorSubcoreMesh(
        core_axis_name="c", subcore_axis_name="s",
        num_cores=NC, num_subcores=NS)
    return pl.kernel(
        _edge_sc_body,
        out_type=jax.ShapeDtypeStruct((NC, NP, DP), jnp.float32),
        mesh=mesh,
        scratch_types=(
            [pltpu.VMEM((2, K), jnp.int32)] * 4
            + [pltpu.VMEM((K, D // 2), jnp.uint32)] * 2
            + [pltpu.VMEM((K, D // 2), jnp.uint32)] * 2
            + [pltpu.VMEM((K, DP), jnp.float32)]
            + [pltpu.VMEM_SHARED((NP, DP), jnp.float32)]
            + [pltpu.SemaphoreType.DMA] * 9
        ),
        compiler_params=pltpu.CompilerParams(
            use_tc_tiling_on_sc=False, needs_layout_passes=False),
    )


def _post_body(ap_ref, sc_ref, a_ref, w_ref, wa_ref, out_ref, *, act):
    aps = ap_ref[0] + ap_ref[1]
    cnt = jnp.maximum(aps[:, D:D + 1], 1.0)
    agg = aps[:, :D] / cnt
    a = a_ref[...]
    z = jnp.concatenate([a[:, j:j + 1] * agg for j in range(DA)], axis=1)
    om = jnp.dot(z, w_ref[...], preferred_element_type=jnp.float32)
    am = jnp.dot(a, wa_ref[...], preferred_element_type=jnp.float32)
    alpha = jnp.sum(agg * am, axis=1, keepdims=True)
    y = sc_ref[...] + alpha * om
    if act:
        y = y * jax.nn.sigmoid(y)
    out_ref[...] = y


def _post_call(aggp, sc, attr, wflat, wa, act):
    return pl.pallas_call(
        functools.partial(_post_body, act=act),
        grid=(N // BN,),
        in_specs=[
            pl.BlockSpec((NC, BN, DP), lambda i: (0, i, 0)),
            pl.BlockSpec((BN, D), lambda i: (i, 0)),
            pl.BlockSpec((BN, DA), lambda i: (i, 0)),
            pl.BlockSpec((D * DA, D), lambda i: (0, 0)),
            pl.BlockSpec((DA, D), lambda i: (0, 0)),
        ],
        out_specs=pl.BlockSpec((BN, D), lambda i: (i, 0)),
        out_shape=jax.ShapeDtypeStruct((N, D), jnp.float32),
    )(aggp, sc, attr, wflat, wa)


def kernel(node_features, node_attr, edge_src, edge_dst, edge_attr,
           edge_scalars,
           W_sc_1, W_lin1_1, fc_w1_1, fc_w2_1, W_alpha_1, W_lin2_1,
           W_sc_2, W_lin1_2, fc_w1_2, fc_w2_2, W_alpha_2, W_lin2_2):
    sd = jnp.stack([edge_src.astype(jnp.int32).reshape(NW, NBLK, K),
                    edge_dst.astype(jnp.int32).reshape(NW, NBLK, K)], axis=2)
    attr = node_attr
    zrows = jnp.zeros((RPT, DP), jnp.float32)
    s_tp = 1.0 / np.sqrt(D * DA)

    x = node_features
    layers = (
        (W_sc_1, W_lin1_1, fc_w1_1, fc_w2_1, W_alpha_1, W_lin2_1, True),
        (W_sc_2, W_lin1_2, fc_w1_2, fc_w2_2, W_alpha_2, W_lin2_2, False),
    )
    for Wsc, Wl1, w1, w2, Wa, Wl2, act in layers:
        wcat = (jnp.concatenate([Wsc, Wl1], axis=2)
                .transpose(1, 0, 2).reshape(D * DA, 2 * D) * s_tp)
        sc, nfp = _pre_call(x, attr, wcat)
        ew = _edge_mlp_call(edge_scalars, edge_attr,
                            w1 / np.sqrt(w1.shape[0]),
                            w2 / np.sqrt(w2.shape[0]))
        aggp = _edge_sc()(nfp, sd, ew, zrows)
        w2f = Wl2.transpose(1, 0, 2).reshape(D * DA, D) * s_tp
        wa = Wa[:, :, 0].T * s_tp
        x = _post_call(aggp, sc, attr, w2f, wa, act)
    return x


# speedup vs baseline: 3.4647x; 1.2793x over previous
import functools

import jax
import jax.numpy as jnp
import numpy as np
from jax import lax
from jax.experimental import pallas as pl
from jax.experimental.pallas import tpu as pltpu
from jax.experimental.pallas import tpu_sc as plsc

N = 10000
E = 320000
D = 128
DA = 8
H = 64
NB = 10
DP = 144
NC = 2
NS = 16
NW = NC * NS
EPW = E // NW
K = 80
NBLK = EPW // K
NP = 10240
RPT = NP // NS

BN = 400
BE = 2560


def _pack2(lo, hi):
    l16 = jax.lax.bitcast_convert_type(lo.astype(jnp.bfloat16), jnp.uint16)
    h16 = jax.lax.bitcast_convert_type(hi.astype(jnp.bfloat16), jnp.uint16)
    return l16.astype(jnp.uint32) | (h16.astype(jnp.uint32) << 16)


def _pre_body(x_ref, a_ref, w_ref, sc_ref, nfp_ref):
    x = x_ref[...]
    a = a_ref[...]
    z = jnp.concatenate([a[:, j:j + 1] * x for j in range(DA)], axis=1)
    r = jnp.dot(z, w_ref[...], preferred_element_type=jnp.float32)
    sc_ref[...] = r[:, :D]
    nf = r[:, D:]
    nfp_ref[...] = _pack2(nf[:, :D // 2], nf[:, D // 2:])


def _pre_call(x, attr, wflat):
    return pl.pallas_call(
        _pre_body,
        grid=(N // BN,),
        in_specs=[
            pl.BlockSpec((BN, D), lambda i: (i, 0)),
            pl.BlockSpec((BN, DA), lambda i: (i, 0)),
            pl.BlockSpec((D * DA, 2 * D), lambda i: (0, 0)),
        ],
        out_specs=[
            pl.BlockSpec((BN, D), lambda i: (i, 0)),
            pl.BlockSpec((BN, D // 2), lambda i: (i, 0)),
        ],
        out_shape=[
            jax.ShapeDtypeStruct((N, D), jnp.float32),
            jax.ShapeDtypeStruct((N, D // 2), jnp.uint32),
        ],
    )(x, attr, wflat)


def _edge_mlp_body(est_ref, w1_ref, w2_ref, out_ref):
    h2 = jax.lax.dot_general(est_ref[...], w1_ref[...],
                             (((0,), (0,)), ((), ())),
                             preferred_element_type=jnp.float32)
    h = h2[:, :H]
    eac = h2[:, H:H + 1]
    h = h * jax.nn.sigmoid(h)
    w = jnp.dot(h, w2_ref[...], preferred_element_type=jnp.float32) * eac
    wr = w.reshape(BE // 16, 2, 8, D)
    lo = wr[:, 0]
    hi = wr[:, 1]
    pl_ = _pack2(lo[..., :D // 2], lo[..., D // 2:])
    ph = _pack2(hi[..., :D // 2], hi[..., D // 2:])
    out_ref[...] = jnp.concatenate([pl_, ph], axis=-1).reshape(BE // 2, D)


def _edge_mlp_call(est, w1a, w2):
    nb1 = est.shape[0]
    return pl.pallas_call(
        _edge_mlp_body,
        grid=(E // BE,),
        in_specs=[
            pl.BlockSpec((nb1, BE), lambda i: (0, i)),
            pl.BlockSpec((nb1, H + 1), lambda i: (0, 0)),
            pl.BlockSpec((H, D), lambda i: (0, 0)),
        ],
        out_specs=pl.BlockSpec((BE // 2, D), lambda i: (i, 0)),
        out_shape=jax.ShapeDtypeStruct((E // 2, D), jnp.uint32),
    )(est, w1a, w2)


def _edge_sc_body(nfp_hbm, sd_hbm, ew_hbm, z_hbm, out_hbm,
                  ii0, ii1, ii2, ii3, rows0, rows1, ewb0, ewb1, srows, acc,
                  is0, is1, is2, is3, gs0, gs1, es0, es1, ss0):
    iidx = (ii0, ii1, ii2, ii3)
    isem = (is0, is1, is2, is3)
    rows = (rows0, rows1)
    ewb = (ewb0, ewb1)
    gs = (gs0, gs1)
    es = (es0, es1)
    c = lax.axis_index("c")
    s = lax.axis_index("s")
    wid = s * NC + c

    pltpu.sync_copy(z_hbm, acc.at[pl.ds(s * RPT, RPT)])
    cpad = jnp.where(lax.iota(jnp.int32, 16) == 0, 1.0, 0.0)

    def preset(k, carry):
        srows[k, pl.ds(D, 16)] = cpad
        return carry

    lax.fori_loop(0, K, preset, 0)
    plsc.subcore_barrier()

    ebase = wid * EPW

    def load_idx(b, r):
        pltpu.async_copy(sd_hbm.at[wid, b], iidx[r], isem[r])

    def wait_idx(r):
        pltpu.make_async_copy(sd_hbm.at[wid, 0], iidx[r], isem[r]).wait()

    def issue_in(b, p, r):
        pltpu.async_copy(nfp_hbm.at[iidx[r].at[0]], rows[p], gs[p])
        pltpu.async_copy(ew_hbm.at[pl.ds((ebase + b * K) // 2, K // 2)],
                         ewb[p], es[p])

    def wait_in(p):
        pltpu.make_async_copy(nfp_hbm.at[ii0.at[0]], rows[p], gs[p]).wait()
        pltpu.make_async_copy(ew_hbm.at[pl.ds(0, K // 2)], ewb[p], es[p]).wait()

    def wait_sc():
        pltpu.make_async_copy(srows, acc.at[ii0.at[1]], ss0).wait()

    def mul(p):
        def body(m, carry):
            for s2 in range(8):
                for j2 in range(D // 32):
                    ka = 16 * m + s2
                    kb = 16 * m + 8 + s2
                    rwa = plsc.bitcast(rows[p][ka, pl.ds(16 * j2, 16)],
                                       jnp.bfloat16)
                    ra0, ra1 = plsc.unpack(
                        rwa, format=plsc.PackFormat.INTERLEAVED)
                    wwa = plsc.bitcast(ewb[p][8 * m + s2, pl.ds(16 * j2, 16)],
                                       jnp.bfloat16)
                    ea0, ea1 = plsc.unpack(
                        wwa, format=plsc.PackFormat.INTERLEAVED)
                    srows[ka, pl.ds(16 * j2, 16)] = ra0 * ea0
                    srows[ka, pl.ds(D // 2 + 16 * j2, 16)] = ra1 * ea1
                    rwb = plsc.bitcast(rows[p][kb, pl.ds(16 * j2, 16)],
                                       jnp.bfloat16)
                    rb0, rb1 = plsc.unpack(
                        rwb, format=plsc.PackFormat.INTERLEAVED)
                    wwb = plsc.bitcast(
                        ewb[p][8 * m + s2, pl.ds(D // 2 + 16 * j2, 16)],
                        jnp.bfloat16)
                    eb0, eb1 = plsc.unpack(
                        wwb, format=plsc.PackFormat.INTERLEAVED)
                    srows[kb, pl.ds(16 * j2, 16)] = rb0 * eb0
                    srows[kb, pl.ds(D // 2 + 16 * j2, 16)] = rb1 * eb1
            return carry
        lax.fori_loop(0, K // 16, body, 0)

    pltpu.sync_copy(sd_hbm.at[wid, 0], ii0)
    issue_in(0, 0, 0)
    load_idx(1, 1)

    def body_step(b, j, notfirst):
        p = j % 2
        q = 1 - p
        wait_in(p)
        wait_idx((j + 1) % 4)
        issue_in(b + 1, q, (j + 1) % 4)
        if notfirst is None:
            wait_sc()
        else:
            @pl.when(notfirst)
            def _():
                wait_sc()
        load_idx(b + 2, (j + 2) % 4)
        mul(p)
        pltpu.async_copy(srows, acc.at[iidx[j].at[1]], ss0, add=True)

    def outer(g, carry):
        for j in range(4):
            b = 4 * g + j
            if j == 0:
                body_step(b, j, g >= 1)
            else:
                body_step(b, j, None)
        return carry

    lax.fori_loop(0, (NBLK - 5) // 4, outer, 0)

    for b in range(NBLK - 5, NBLK):
        j = b % 4
        p = j % 2
        wait_in(p)
        if b + 1 < NBLK:
            wait_idx((j + 1) % 4)
            issue_in(b + 1, 1 - p, (j + 1) % 4)
        wait_sc()
        if b + 2 < NBLK:
            load_idx(b + 2, (j + 2) % 4)
        mul(p)
        pltpu.async_copy(srows, acc.at[iidx[j].at[1]], ss0, add=True)
    wait_sc()
    plsc.subcore_barrier()

    rbase = s * RPT
    pltpu.sync_copy(acc.at[pl.ds(rbase, RPT)], out_hbm.at[c, pl.ds(rbase, RPT)])


@functools.cache
def _edge_sc():
    mesh = plsc.VectorSubcoreMesh(
        core_axis_name="c", subcore_axis_name="s",
        num_cores=NC, num_subcores=NS)
    return pl.kernel(
        _edge_sc_body,
        out_type=jax.ShapeDtypeStruct((NC, NP, DP), jnp.float32),
        mesh=mesh,
        scratch_types=(
            [pltpu.VMEM((2, K), jnp.int32)] * 4
            + [pltpu.VMEM((K, D // 2), jnp.uint32)] * 2
            + [pltpu.VMEM((K // 2, D), jnp.uint32)] * 2
            + [pltpu.VMEM((K, DP), jnp.float32)]
            + [pltpu.VMEM_SHARED((NP, DP), jnp.float32)]
            + [pltpu.SemaphoreType.DMA] * 9
        ),
        compiler_params=pltpu.CompilerParams(
            use_tc_tiling_on_sc=False, needs_layout_passes=False),
    )


def _post_body(ap_ref, sc_ref, a_ref, w_ref, wa_ref, out_ref, *, act):
    aps = ap_ref[0] + ap_ref[1]
    cnt = jnp.maximum(aps[:, D:D + 1], 1.0)
    agg = aps[:, :D] / cnt
    a = a_ref[...]
    z = jnp.concatenate([a[:, j:j + 1] * agg for j in range(DA)], axis=1)
    om = jnp.dot(z, w_ref[...], preferred_element_type=jnp.float32)
    am = jnp.dot(a, wa_ref[...], preferred_element_type=jnp.float32)
    alpha = jnp.sum(agg * am, axis=1, keepdims=True)
    y = sc_ref[...] + alpha * om
    if act:
        y = y * jax.nn.sigmoid(y)
    out_ref[...] = y


def _post_call(aggp, sc, attr, wflat, wa, act):
    return pl.pallas_call(
        functools.partial(_post_body, act=act),
        grid=(N // BN,),
        in_specs=[
            pl.BlockSpec((NC, BN, DP), lambda i: (0, i, 0)),
            pl.BlockSpec((BN, D), lambda i: (i, 0)),
            pl.BlockSpec((BN, DA), lambda i: (i, 0)),
            pl.BlockSpec((D * DA, D), lambda i: (0, 0)),
            pl.BlockSpec((DA, D), lambda i: (0, 0)),
        ],
        out_specs=pl.BlockSpec((BN, D), lambda i: (i, 0)),
        out_shape=jax.ShapeDtypeStruct((N, D), jnp.float32),
    )(aggp, sc, attr, wflat, wa)


def kernel(node_features, node_attr, edge_src, edge_dst, edge_attr,
           edge_scalars,
           W_sc_1, W_lin1_1, fc_w1_1, fc_w2_1, W_alpha_1, W_lin2_1,
           W_sc_2, W_lin1_2, fc_w1_2, fc_w2_2, W_alpha_2, W_lin2_2):
    sd = jnp.stack([edge_src.astype(jnp.int32).reshape(NW, NBLK, K),
                    edge_dst.astype(jnp.int32).reshape(NW, NBLK, K)], axis=2)
    attr = node_attr
    zrows = jnp.zeros((RPT, DP), jnp.float32)
    s_tp = 1.0 / np.sqrt(D * DA)

    x = node_features
    layers = (
        (W_sc_1, W_lin1_1, fc_w1_1, fc_w2_1, W_alpha_1, W_lin2_1, True),
        (W_sc_2, W_lin1_2, fc_w1_2, fc_w2_2, W_alpha_2, W_lin2_2, False),
    )
    est = jnp.concatenate([edge_scalars.T, edge_attr.T], axis=0)
    for Wsc, Wl1, w1, w2, Wa, Wl2, act in layers:
        wcat = (jnp.concatenate([Wsc, Wl1], axis=2)
                .transpose(1, 0, 2).reshape(D * DA, 2 * D) * s_tp)
        sc, nfp = _pre_call(x, attr, wcat)
        w1a = (jnp.zeros((NB + 1, H + 1), jnp.float32)
               .at[:NB, :H].set(w1 / np.sqrt(w1.shape[0]))
               .at[NB, H].set(1.0))
        ew = _edge_mlp_call(est, w1a, w2 / np.sqrt(w2.shape[0]))
        aggp = _edge_sc()(nfp, sd, ew, zrows)
        w2f = Wl2.transpose(1, 0, 2).reshape(D * DA, D) * s_tp
        wa = Wa[:, :, 0].T * s_tp
        x = _post_call(aggp, sc, attr, w2f, wa, act)
    return x


# speedup vs baseline: 3.6122x; 1.0426x over previous
import functools

import jax
import jax.numpy as jnp
import numpy as np
from jax import lax
from jax.experimental import pallas as pl
from jax.experimental.pallas import tpu as pltpu
from jax.experimental.pallas import tpu_sc as plsc

N = 10000
E = 320000
D = 128
DA = 8
H = 64
NB = 10
DP = 144
NC = 2
NS = 16
NW = NC * NS
EPW = E // NW
K = 80
NBLK = EPW // K
NP = 10240
RPT = NP // NS

BN = 400
BE = 6400


def _pack2(lo, hi):
    l16 = jax.lax.bitcast_convert_type(lo.astype(jnp.bfloat16), jnp.uint16)
    h16 = jax.lax.bitcast_convert_type(hi.astype(jnp.bfloat16), jnp.uint16)
    return l16.astype(jnp.uint32) | (h16.astype(jnp.uint32) << 16)


def _pre_body(x_ref, a_ref, w_ref, sc_ref, nfp_ref):
    x = x_ref[...]
    a = a_ref[...]
    z = jnp.concatenate([a[:, j:j + 1] * x for j in range(DA)], axis=1)
    r = jnp.dot(z, w_ref[...], preferred_element_type=jnp.float32)
    sc_ref[...] = r[:, :D]
    nf = r[:, D:]
    nfp_ref[...] = _pack2(nf[:, :D // 2], nf[:, D // 2:])


def _pre_call(x, attr, wflat):
    return pl.pallas_call(
        _pre_body,
        grid=(N // BN,),
        in_specs=[
            pl.BlockSpec((BN, D), lambda i: (i, 0)),
            pl.BlockSpec((BN, DA), lambda i: (i, 0)),
            pl.BlockSpec((D * DA, 2 * D), lambda i: (0, 0)),
        ],
        out_specs=[
            pl.BlockSpec((BN, D), lambda i: (i, 0)),
            pl.BlockSpec((BN, D // 2), lambda i: (i, 0)),
        ],
        out_shape=[
            jax.ShapeDtypeStruct((N, D), jnp.float32),
            jax.ShapeDtypeStruct((N, D // 2), jnp.uint32),
        ],
    )(x, attr, wflat)


def _edge_mlp_body(est_ref, w1_ref, w2_ref, out_ref):
    h2 = jax.lax.dot_general(est_ref[...], w1_ref[...],
                             (((0,), (0,)), ((), ())),
                             preferred_element_type=jnp.float32)
    h = h2[:, :H]
    eac = h2[:, H:H + 1]
    h = h * jax.nn.sigmoid(h)
    w = jnp.dot(h, w2_ref[...], preferred_element_type=jnp.float32) * eac
    wr = w.reshape(BE // 16, 2, 8, D)
    lo = wr[:, 0]
    hi = wr[:, 1]
    pl_ = _pack2(lo[..., :D // 2], lo[..., D // 2:])
    ph = _pack2(hi[..., :D // 2], hi[..., D // 2:])
    out_ref[...] = jnp.concatenate([pl_, ph], axis=-1).reshape(BE // 2, D)


def _edge_mlp_call(est, w1a, w2):
    nb1 = est.shape[0]
    return pl.pallas_call(
        _edge_mlp_body,
        grid=(E // BE,),
        in_specs=[
            pl.BlockSpec((nb1, BE), lambda i: (0, i)),
            pl.BlockSpec((nb1, H + 1), lambda i: (0, 0)),
            pl.BlockSpec((H, D), lambda i: (0, 0)),
        ],
        out_specs=pl.BlockSpec((BE // 2, D), lambda i: (i, 0)),
        out_shape=jax.ShapeDtypeStruct((E // 2, D), jnp.uint32),
    )(est, w1a, w2)


def _edge_sc_body(nfp_hbm, src_hbm, dst_hbm, ew_hbm, z_hbm, out_hbm,
                  sb0, sb1, sb2, sb3, db0, db1, db2, db3,
                  rows0, rows1, ewb0, ewb1, srows, acc,
                  is0, is1, is2, is3, gs0, gs1, es0, es1, ss0):
    sbuf = (sb0, sb1, sb2, sb3)
    dbuf = (db0, db1, db2, db3)
    isem = (is0, is1, is2, is3)
    rows = (rows0, rows1)
    ewb = (ewb0, ewb1)
    gs = (gs0, gs1)
    es = (es0, es1)
    c = lax.axis_index("c")
    s = lax.axis_index("s")
    wid = s * NC + c

    pltpu.sync_copy(z_hbm, acc.at[pl.ds(s * RPT, RPT)])
    cpad = jnp.where(lax.iota(jnp.int32, 16) == 0, 1.0, 0.0)

    def preset(k, carry):
        srows[k, pl.ds(D, 16)] = cpad
        return carry

    lax.fori_loop(0, K, preset, 0)
    plsc.subcore_barrier()

    ebase = wid * EPW

    def load_idx(b, r):
        pltpu.async_copy(src_hbm.at[pl.ds(ebase + b * K, K)], sbuf[r], isem[r])
        pltpu.async_copy(dst_hbm.at[pl.ds(ebase + b * K, K)], dbuf[r], isem[r])

    def wait_idx(r):
        pltpu.make_async_copy(src_hbm.at[pl.ds(0, K)], sbuf[r], isem[r]).wait()
        pltpu.make_async_copy(dst_hbm.at[pl.ds(0, K)], dbuf[r], isem[r]).wait()

    def issue_in(b, p, r):
        pltpu.async_copy(nfp_hbm.at[sbuf[r]], rows[p], gs[p])
        pltpu.async_copy(ew_hbm.at[pl.ds((ebase + b * K) // 2, K // 2)],
                         ewb[p], es[p])

    def wait_in(p):
        pltpu.make_async_copy(nfp_hbm.at[sb0], rows[p], gs[p]).wait()
        pltpu.make_async_copy(ew_hbm.at[pl.ds(0, K // 2)], ewb[p], es[p]).wait()

    def wait_sc():
        pltpu.make_async_copy(srows, acc.at[db0], ss0).wait()

    def mul(p):
        def body(m, carry):
            for s2 in range(8):
                for j2 in range(D // 32):
                    ka = 16 * m + s2
                    kb = 16 * m + 8 + s2
                    pa = (plsc.bitcast(rows[p][ka, pl.ds(16 * j2, 16)],
                                       jnp.bfloat16)
                          * plsc.bitcast(ewb[p][8 * m + s2,
                                                pl.ds(16 * j2, 16)],
                                         jnp.bfloat16))
                    a0, a1 = plsc.unpack(
                        pa, format=plsc.PackFormat.INTERLEAVED)
                    srows[ka, pl.ds(16 * j2, 16)] = a0
                    srows[ka, pl.ds(D // 2 + 16 * j2, 16)] = a1
                    pb = (plsc.bitcast(rows[p][kb, pl.ds(16 * j2, 16)],
                                       jnp.bfloat16)
                          * plsc.bitcast(ewb[p][8 * m + s2,
                                                pl.ds(D // 2 + 16 * j2, 16)],
                                         jnp.bfloat16))
                    b0, b1 = plsc.unpack(
                        pb, format=plsc.PackFormat.INTERLEAVED)
                    srows[kb, pl.ds(16 * j2, 16)] = b0
                    srows[kb, pl.ds(D // 2 + 16 * j2, 16)] = b1
            return carry
        lax.fori_loop(0, K // 16, body, 0)

    pltpu.sync_copy(src_hbm.at[pl.ds(ebase, K)], sb0)
    pltpu.sync_copy(dst_hbm.at[pl.ds(ebase, K)], db0)
    issue_in(0, 0, 0)
    load_idx(1, 1)

    def body_step(b, j, notfirst):
        p = j % 2
        q = 1 - p
        wait_in(p)
        wait_idx((j + 1) % 4)
        issue_in(b + 1, q, (j + 1) % 4)
        if notfirst is None:
            wait_sc()
        else:
            @pl.when(notfirst)
            def _():
                wait_sc()
        load_idx(b + 2, (j + 2) % 4)
        mul(p)
        pltpu.async_copy(srows, acc.at[dbuf[j]], ss0, add=True)

    def outer(g, carry):
        for j in range(4):
            b = 4 * g + j
            if j == 0:
                body_step(b, j, g >= 1)
            else:
                body_step(b, j, None)
        return carry

    lax.fori_loop(0, (NBLK - 5) // 4, outer, 0)

    for b in range(NBLK - 5, NBLK):
        j = b % 4
        p = j % 2
        wait_in(p)
        if b + 1 < NBLK:
            wait_idx((j + 1) % 4)
            issue_in(b + 1, 1 - p, (j + 1) % 4)
        wait_sc()
        if b + 2 < NBLK:
            load_idx(b + 2, (j + 2) % 4)
        mul(p)
        pltpu.async_copy(srows, acc.at[dbuf[j]], ss0, add=True)
    wait_sc()
    plsc.subcore_barrier()

    rbase = s * RPT
    pltpu.sync_copy(acc.at[pl.ds(rbase, RPT)], out_hbm.at[c, pl.ds(rbase, RPT)])


@functools.cache
def _edge_sc():
    mesh = plsc.VectorSubcoreMesh(
        core_axis_name="c", subcore_axis_name="s",
        num_cores=NC, num_subcores=NS)
    return pl.kernel(
        _edge_sc_body,
        out_type=jax.ShapeDtypeStruct((NC, NP, DP), jnp.float32),
        mesh=mesh,
        scratch_types=(
            [pltpu.VMEM((K,), jnp.int32)] * 8
            + [pltpu.VMEM((K, D // 2), jnp.uint32)] * 2
            + [pltpu.VMEM((K // 2, D), jnp.uint32)] * 2
            + [pltpu.VMEM((K, DP), jnp.float32)]
            + [pltpu.VMEM_SHARED((NP, DP), jnp.float32)]
            + [pltpu.SemaphoreType.DMA] * 9
        ),
        compiler_params=pltpu.CompilerParams(
            use_tc_tiling_on_sc=False, needs_layout_passes=False),
    )


def _post_body(ap_ref, sc_ref, a_ref, w_ref, wa_ref, out_ref, *, act):
    aps = ap_ref[0] + ap_ref[1]
    cnt = jnp.maximum(aps[:, D:D + 1], 1.0)
    agg = aps[:, :D] / cnt
    a = a_ref[...]
    z = jnp.concatenate([a[:, j:j + 1] * agg for j in range(DA)], axis=1)
    om = jnp.dot(z, w_ref[...], preferred_element_type=jnp.float32)
    am = jnp.dot(a, wa_ref[...], preferred_element_type=jnp.float32)
    alpha = jnp.sum(agg * am, axis=1, keepdims=True)
    y = sc_ref[...] + alpha * om
    if act:
        y = y * jax.nn.sigmoid(y)
    out_ref[...] = y


def _post_call(aggp, sc, attr, wflat, wa, act):
    return pl.pallas_call(
        functools.partial(_post_body, act=act),
        grid=(N // BN,),
        in_specs=[
            pl.BlockSpec((NC, BN, DP), lambda i: (0, i, 0)),
            pl.BlockSpec((BN, D), lambda i: (i, 0)),
            pl.BlockSpec((BN, DA), lambda i: (i, 0)),
            pl.BlockSpec((D * DA, D), lambda i: (0, 0)),
            pl.BlockSpec((DA, D), lambda i: (0, 0)),
        ],
        out_specs=pl.BlockSpec((BN, D), lambda i: (i, 0)),
        out_shape=jax.ShapeDtypeStruct((N, D), jnp.float32),
    )(aggp, sc, attr, wflat, wa)


def kernel(node_features, node_attr, edge_src, edge_dst, edge_attr,
           edge_scalars,
           W_sc_1, W_lin1_1, fc_w1_1, fc_w2_1, W_alpha_1, W_lin2_1,
           W_sc_2, W_lin1_2, fc_w1_2, fc_w2_2, W_alpha_2, W_lin2_2):
    esrc = edge_src.astype(jnp.int32)
    edst = edge_dst.astype(jnp.int32)
    attr = node_attr
    zrows = jnp.zeros((RPT, DP), jnp.float32)
    s_tp = 1.0 / np.sqrt(D * DA)

    x = node_features
    layers = (
        (W_sc_1, W_lin1_1, fc_w1_1, fc_w2_1, W_alpha_1, W_lin2_1, True),
        (W_sc_2, W_lin1_2, fc_w1_2, fc_w2_2, W_alpha_2, W_lin2_2, False),
    )
    est = jnp.concatenate([edge_scalars.T, edge_attr.T], axis=0)
    for Wsc, Wl1, w1, w2, Wa, Wl2, act in layers:
        wcat = (jnp.concatenate([Wsc, Wl1], axis=2)
                .transpose(1, 0, 2).reshape(D * DA, 2 * D) * s_tp)
        sc, nfp = _pre_call(x, attr, wcat)
        w1a = (jnp.zeros((NB + 1, H + 1), jnp.float32)
               .at[:NB, :H].set(w1 / np.sqrt(w1.shape[0]))
               .at[NB, H].set(1.0))
        ew = _edge_mlp_call(est, w1a, w2 / np.sqrt(w2.shape[0]))
        aggp = _edge_sc()(nfp, esrc, edst, ew, zrows)
        w2f = Wl2.transpose(1, 0, 2).reshape(D * DA, D) * s_tp
        wa = Wa[:, :, 0].T * s_tp
        x = _post_call(aggp, sc, attr, w2f, wa, act)
    return x


# speedup vs baseline: 3.6133x; 1.0003x over previous
import functools

import jax
import jax.numpy as jnp
import numpy as np
from jax import lax
from jax.experimental import pallas as pl
from jax.experimental.pallas import tpu as pltpu
from jax.experimental.pallas import tpu_sc as plsc

N = 10000
E = 320000
D = 128
DA = 8
H = 64
NB = 10
DP = 144
NC = 2
NS = 16
NW = NC * NS
EPW = E // NW
K = 80
NBLK = EPW // K
NP = 10240
RPT = NP // NS

BN = 400
BE = 6400


def _pack2(lo, hi):
    l16 = jax.lax.bitcast_convert_type(lo.astype(jnp.bfloat16), jnp.uint16)
    h16 = jax.lax.bitcast_convert_type(hi.astype(jnp.bfloat16), jnp.uint16)
    return l16.astype(jnp.uint32) | (h16.astype(jnp.uint32) << 16)


def _pre_body(x_ref, a_ref, w_ref, sc_ref, nfp_ref):
    x = x_ref[...]
    a = a_ref[...]
    z = jnp.concatenate([a[:, j:j + 1] * x for j in range(DA)], axis=1)
    r = jnp.dot(z, w_ref[...], preferred_element_type=jnp.float32)
    sc_ref[...] = r[:, :D]
    nf = r[:, D:]
    nfp_ref[...] = _pack2(nf[:, :D // 2], nf[:, D // 2:])


def _pre_call(x, attr, wflat):
    return pl.pallas_call(
        _pre_body,
        grid=(N // BN,),
        in_specs=[
            pl.BlockSpec((BN, D), lambda i: (i, 0)),
            pl.BlockSpec((BN, DA), lambda i: (i, 0)),
            pl.BlockSpec((D * DA, 2 * D), lambda i: (0, 0)),
        ],
        out_specs=[
            pl.BlockSpec((BN, D), lambda i: (i, 0)),
            pl.BlockSpec((BN, D // 2), lambda i: (i, 0)),
        ],
        out_shape=[
            jax.ShapeDtypeStruct((N, D), jnp.float32),
            jax.ShapeDtypeStruct((N, D // 2), jnp.uint32),
        ],
    )(x, attr, wflat)


def _edge_mlp_body(est_ref, w1_ref, w2_ref, out_ref):
    h2 = jax.lax.dot_general(est_ref[...], w1_ref[...],
                             (((0,), (0,)), ((), ())),
                             preferred_element_type=jnp.float32)
    h = h2[:, :H]
    eac = h2[:, H:H + 1]
    h = h * jax.nn.sigmoid(h)
    w = jnp.dot(h.astype(jnp.bfloat16), w2_ref[...],
                preferred_element_type=jnp.float32) * eac
    wr = w.reshape(BE // 16, 2, 8, D)
    lo = wr[:, 0]
    hi = wr[:, 1]
    pl_ = _pack2(lo[..., :D // 2], lo[..., D // 2:])
    ph = _pack2(hi[..., :D // 2], hi[..., D // 2:])
    out_ref[...] = jnp.concatenate([pl_, ph], axis=-1).reshape(BE // 2, D)


def _edge_mlp_call(est, w1a, w2):
    nb1 = est.shape[0]
    return pl.pallas_call(
        _edge_mlp_body,
        grid=(E // BE,),
        in_specs=[
            pl.BlockSpec((nb1, BE), lambda i: (0, i)),
            pl.BlockSpec((nb1, H + 1), lambda i: (0, 0)),
            pl.BlockSpec((H, D), lambda i: (0, 0)),
        ],
        out_specs=pl.BlockSpec((BE // 2, D), lambda i: (i, 0)),
        out_shape=jax.ShapeDtypeStruct((E // 2, D), jnp.uint32),
    )(est, w1a, w2)


def _edge_sc_body(nfp_hbm, src_hbm, dst_hbm, ew_hbm, z_hbm, out_hbm,
                  sb0, sb1, sb2, sb3, db0, db1, db2, db3,
                  rows0, rows1, ewb0, ewb1, srows, acc,
                  is0, is1, is2, is3, gs0, gs1, es0, es1, ss0):
    sbuf = (sb0, sb1, sb2, sb3)
    dbuf = (db0, db1, db2, db3)
    isem = (is0, is1, is2, is3)
    rows = (rows0, rows1)
    ewb = (ewb0, ewb1)
    gs = (gs0, gs1)
    es = (es0, es1)
    c = lax.axis_index("c")
    s = lax.axis_index("s")
    wid = s * NC + c

    pltpu.sync_copy(z_hbm, acc.at[pl.ds(s * RPT, RPT)])
    cpad = jnp.where(lax.iota(jnp.int32, 16) == 0, 1.0, 0.0)

    def preset(k, carry):
        srows[k, pl.ds(D, 16)] = cpad
        return carry

    lax.fori_loop(0, K, preset, 0)
    plsc.subcore_barrier()

    ebase = wid * EPW

    def load_idx(b, r):
        pltpu.async_copy(src_hbm.at[pl.ds(ebase + b * K, K)], sbuf[r], isem[r])
        pltpu.async_copy(dst_hbm.at[pl.ds(ebase + b * K, K)], dbuf[r], isem[r])

    def wait_idx(r):
        pltpu.make_async_copy(src_hbm.at[pl.ds(0, K)], sbuf[r], isem[r]).wait()
        pltpu.make_async_copy(dst_hbm.at[pl.ds(0, K)], dbuf[r], isem[r]).wait()

    def issue_in(b, p, r):
        pltpu.async_copy(nfp_hbm.at[sbuf[r]], rows[p], gs[p])
        pltpu.async_copy(ew_hbm.at[pl.ds((ebase + b * K) // 2, K // 2)],
                         ewb[p], es[p])

    def wait_in(p):
        pltpu.make_async_copy(nfp_hbm.at[sb0], rows[p], gs[p]).wait()
        pltpu.make_async_copy(ew_hbm.at[pl.ds(0, K // 2)], ewb[p], es[p]).wait()

    def wait_sc():
        pltpu.make_async_copy(srows, acc.at[db0], ss0).wait()

    def mul(p):
        def body(m, carry):
            for s2 in range(8):
                for j2 in range(D // 32):
                    ka = 16 * m + s2
                    kb = 16 * m + 8 + s2
                    pa = (plsc.bitcast(rows[p][ka, pl.ds(16 * j2, 16)],
                                       jnp.bfloat16)
                          * plsc.bitcast(ewb[p][8 * m + s2,
                                                pl.ds(16 * j2, 16)],
                                         jnp.bfloat16))
                    a0, a1 = plsc.unpack(
                        pa, format=plsc.PackFormat.INTERLEAVED)
                    srows[ka, pl.ds(16 * j2, 16)] = a0
                    srows[ka, pl.ds(D // 2 + 16 * j2, 16)] = a1
                    pb = (plsc.bitcast(rows[p][kb, pl.ds(16 * j2, 16)],
                                       jnp.bfloat16)
                          * plsc.bitcast(ewb[p][8 * m + s2,
                                                pl.ds(D // 2 + 16 * j2, 16)],
                                         jnp.bfloat16))
                    b0, b1 = plsc.unpack(
                        pb, format=plsc.PackFormat.INTERLEAVED)
                    srows[kb, pl.ds(16 * j2, 16)] = b0
                    srows[kb, pl.ds(D // 2 + 16 * j2, 16)] = b1
            return carry
        lax.fori_loop(0, K // 16, body, 0)

    pltpu.sync_copy(src_hbm.at[pl.ds(ebase, K)], sb0)
    pltpu.sync_copy(dst_hbm.at[pl.ds(ebase, K)], db0)
    issue_in(0, 0, 0)
    load_idx(1, 1)

    def body_step(b, j, notfirst):
        p = j % 2
        q = 1 - p
        wait_in(p)
        wait_idx((j + 1) % 4)
        issue_in(b + 1, q, (j + 1) % 4)
        if notfirst is None:
            wait_sc()
        else:
            @pl.when(notfirst)
            def _():
                wait_sc()
        load_idx(b + 2, (j + 2) % 4)
        mul(p)
        pltpu.async_copy(srows, acc.at[dbuf[j]], ss0, add=True)

    def outer(g, carry):
        for j in range(4):
            b = 4 * g + j
            if j == 0:
                body_step(b, j, g >= 1)
            else:
                body_step(b, j, None)
        return carry

    lax.fori_loop(0, (NBLK - 5) // 4, outer, 0)

    for b in range(NBLK - 5, NBLK):
        j = b % 4
        p = j % 2
        wait_in(p)
        if b + 1 < NBLK:
            wait_idx((j + 1) % 4)
            issue_in(b + 1, 1 - p, (j + 1) % 4)
        wait_sc()
        if b + 2 < NBLK:
            load_idx(b + 2, (j + 2) % 4)
        mul(p)
        pltpu.async_copy(srows, acc.at[dbuf[j]], ss0, add=True)
    wait_sc()
    plsc.subcore_barrier()

    rbase = s * RPT
    pltpu.sync_copy(acc.at[pl.ds(rbase, RPT)], out_hbm.at[c, pl.ds(rbase, RPT)])


@functools.cache
def _edge_sc():
    mesh = plsc.VectorSubcoreMesh(
        core_axis_name="c", subcore_axis_name="s",
        num_cores=NC, num_subcores=NS)
    return pl.kernel(
        _edge_sc_body,
        out_type=jax.ShapeDtypeStruct((NC, NP, DP), jnp.float32),
        mesh=mesh,
        scratch_types=(
            [pltpu.VMEM((K,), jnp.int32)] * 8
            + [pltpu.VMEM((K, D // 2), jnp.uint32)] * 2
            + [pltpu.VMEM((K // 2, D), jnp.uint32)] * 2
            + [pltpu.VMEM((K, DP), jnp.float32)]
            + [pltpu.VMEM_SHARED((NP, DP), jnp.float32)]
            + [pltpu.SemaphoreType.DMA] * 9
        ),
        compiler_params=pltpu.CompilerParams(
            use_tc_tiling_on_sc=False, needs_layout_passes=False),
    )


def _post_body(ap_ref, sc_ref, a_ref, w_ref, wa_ref, out_ref, *, act):
    aps = ap_ref[0] + ap_ref[1]
    cnt = jnp.maximum(aps[:, D:D + 1], 1.0)
    agg = aps[:, :D] / cnt
    a = a_ref[...]
    z = jnp.concatenate([a[:, j:j + 1] * agg for j in range(DA)], axis=1)
    om = jnp.dot(z, w_ref[...], preferred_element_type=jnp.float32)
    am = jnp.dot(a, wa_ref[...], preferred_element_type=jnp.float32)
    alpha = jnp.sum(agg * am, axis=1, keepdims=True)
    y = sc_ref[...] + alpha * om
    if act:
        y = y * jax.nn.sigmoid(y)
    out_ref[...] = y


def _post_call(aggp, sc, attr, wflat, wa, act):
    return pl.pallas_call(
        functools.partial(_post_body, act=act),
        grid=(N // BN,),
        in_specs=[
            pl.BlockSpec((NC, BN, DP), lambda i: (0, i, 0)),
            pl.BlockSpec((BN, D), lambda i: (i, 0)),
            pl.BlockSpec((BN, DA), lambda i: (i, 0)),
            pl.BlockSpec((D * DA, D), lambda i: (0, 0)),
            pl.BlockSpec((DA, D), lambda i: (0, 0)),
        ],
        out_specs=pl.BlockSpec((BN, D), lambda i: (i, 0)),
        out_shape=jax.ShapeDtypeStruct((N, D), jnp.float32),
    )(aggp, sc, attr, wflat, wa)


def kernel(node_features, node_attr, edge_src, edge_dst, edge_attr,
           edge_scalars,
           W_sc_1, W_lin1_1, fc_w1_1, fc_w2_1, W_alpha_1, W_lin2_1,
           W_sc_2, W_lin1_2, fc_w1_2, fc_w2_2, W_alpha_2, W_lin2_2):
    esrc = edge_src.astype(jnp.int32)
    edst = edge_dst.astype(jnp.int32)
    attr = node_attr
    zrows = jnp.zeros((RPT, DP), jnp.float32)
    s_tp = 1.0 / np.sqrt(D * DA)

    x = node_features
    layers = (
        (W_sc_1, W_lin1_1, fc_w1_1, fc_w2_1, W_alpha_1, W_lin2_1, True),
        (W_sc_2, W_lin1_2, fc_w1_2, fc_w2_2, W_alpha_2, W_lin2_2, False),
    )
    est = jnp.concatenate([edge_scalars.T, edge_attr.T], axis=0)
    for Wsc, Wl1, w1, w2, Wa, Wl2, act in layers:
        wcat = (jnp.concatenate([Wsc, Wl1], axis=2)
                .transpose(1, 0, 2).reshape(D * DA, 2 * D) * s_tp)
        sc, nfp = _pre_call(x, attr, wcat)
        w1a = (jnp.zeros((NB + 1, H + 1), jnp.float32)
               .at[:NB, :H].set(w1 / np.sqrt(w1.shape[0]))
               .at[NB, H].set(1.0))
        ew = _edge_mlp_call(
            est, w1a, (w2 / np.sqrt(w2.shape[0])).astype(jnp.bfloat16))
        aggp = _edge_sc()(nfp, esrc, edst, ew, zrows)
        w2f = Wl2.transpose(1, 0, 2).reshape(D * DA, D) * s_tp
        wa = Wa[:, :, 0].T * s_tp
        x = _post_call(aggp, sc, attr, w2f, wa, act)
    return x
